# baseline (speedup 1.0000x reference)
import jax
import jax.numpy as jnp
from jax import lax
from jax.experimental import pallas as pl
from jax.experimental.pallas import tpu as pltpu
from jax.experimental.pallas import tpu_sc as plsc

NUM_FEATURES = 26
BATCH = 4096
VOCAB = 100000
EMB_DIM = 32

NC = 2
NS = 16
LANES = 16
NW = NC * NS
CHUNK = BATCH // NW

NTILES = 782
NTILES_USED = 782
TOTAL_TC = NUM_FEATURES * NTILES
PER_W = [TOTAL_TC // NW + (1 if w < TOTAL_TC % NW else 0) for w in range(NW)]
LO_W = [0] * NW
for w in range(1, NW):
    LO_W[w] = LO_W[w - 1] + PER_W[w - 1]


def _relayout_body(tab_hbm, lin_hbm, stage_v, col_v, sem, osem):
    c = lax.axis_index("c")
    s = lax.axis_index("s")
    wid = s * NC + c

    lo = jnp.int32(0)
    hi = jnp.int32(0)
    for w in range(NW):
        lo = jnp.where(wid == w, jnp.int32(LO_W[w]), lo)
        hi = jnp.where(wid == w, jnp.int32(LO_W[w] + PER_W[w]), hi)

    iota = lax.iota(jnp.int32, LANES)

    def fetch(tc, buf):
        f = tc // NTILES
        t = tc % NTILES
        toff = pl.multiple_of(t * 128, 128)
        for g in range(4):
            pltpu.async_copy(
                tab_hbm.at[f, pl.ds(8 * g, 8), pl.ds(toff, 128)],
                stage_v.at[buf, pl.ds(8 * g, 8)],
                sem,
            )

    def wait_fetch(tc, buf):
        f = tc // NTILES
        t = tc % NTILES
        toff = pl.multiple_of(t * 128, 128)
        for g in range(4):
            pltpu.make_async_copy(
                tab_hbm.at[f, pl.ds(8 * g, 8), pl.ds(toff, 128)],
                stage_v.at[buf, pl.ds(8 * g, 8)],
                sem,
            ).wait()

    fetch(lo, 0)

    def body(i, _):
        tc = lo + i
        buf = i % 2

        @pl.when(tc + 1 < hi)
        def _():
            fetch(tc + 1, 1 - buf)

        wait_fetch(tc, buf)

        for j in range(32):
            for k in range(8):
                g = plsc.load_gather(
                    stage_v,
                    [
                        jnp.full((LANES,), buf, dtype=jnp.int32),
                        iota + 16 * (k % 2),
                        jnp.full((LANES,), 4 * j + k // 2, dtype=jnp.int32),
                    ],
                )
                col_v[j, pl.ds(16 * k, 16)] = g

        f = tc // NTILES
        t = tc % NTILES
        r0 = pl.multiple_of(f * (VOCAB // 4) + 32 * t, 8)

        @pl.when(t < NTILES - 1)
        def _():
            pltpu.sync_copy(col_v, lin_hbm.at[pl.ds(r0, 32), :])

        @pl.when(t == NTILES - 1)
        def _():
            pltpu.sync_copy(
                col_v.at[pl.ds(0, 8), :], lin_hbm.at[pl.ds(r0, 8), :]
            )

        return 0

    lax.fori_loop(0, hi - lo, body, 0, unroll=False)


@jax.jit
def _relayout(tabT):
    mesh = plsc.VectorSubcoreMesh(
        core_axis_name="c", subcore_axis_name="s", num_cores=NC, num_subcores=NS
    )
    k = pl.kernel(
        _relayout_body,
        out_type=jax.ShapeDtypeStruct((VOCAB * NUM_FEATURES // 4, 128), jnp.float32),
        mesh=mesh,
        scratch_types=[
            pltpu.VMEM((2, EMB_DIM, 128), jnp.float32),
            pltpu.VMEM((EMB_DIM, 128), jnp.float32),
            pltpu.SemaphoreType.DMA,
            pltpu.SemaphoreType.DMA,
        ],
        compiler_params=pltpu.CompilerParams(
            use_tc_tiling_on_sc=True, needs_layout_passes=False
        ),
    )
    return k(tabT)


def _gather_body(idx_hbm, tab_hbm, out_hbm, idx_v, gidx_v, rows_v, sem):
    c = lax.axis_index("c")
    s = lax.axis_index("s")
    wid = s * NC + c
    base_b = wid * CHUNK

    pltpu.sync_copy(idx_hbm.at[:, pl.ds(base_b, CHUNK)], idx_v)

    for f in range(NUM_FEATURES):
        off = jnp.full((LANES,), f * VOCAB, dtype=jnp.int32)
        for k in range(CHUNK // LANES):
            sl = pl.ds(k * LANES, LANES)
            gidx_v[f, sl] = idx_v[f, sl] + off

    copies = [None, None]
    for f in range(NUM_FEATURES):
        b = f % 2
        if copies[b] is not None:
            copies[b].wait()
            pltpu.sync_copy(
                rows_v.at[b],
                out_hbm.at[pl.ds((f - 2) * BATCH + base_b, CHUNK)],
            )
        copies[b] = pltpu.async_copy(tab_hbm.at[gidx_v.at[f]], rows_v.at[b], sem)
    for f in (NUM_FEATURES - 2, NUM_FEATURES - 1):
        b = f % 2
        copies[b].wait()
        pltpu.sync_copy(
            rows_v.at[b],
            out_hbm.at[pl.ds(f * BATCH + base_b, CHUNK)],
        )


@jax.jit
def _gather(idx32, lin_tables):
    mesh = plsc.VectorSubcoreMesh(
        core_axis_name="c", subcore_axis_name="s", num_cores=NC, num_subcores=NS
    )
    k = pl.kernel(
        _gather_body,
        out_type=jax.ShapeDtypeStruct((NUM_FEATURES * BATCH, EMB_DIM), jnp.float32),
        mesh=mesh,
        scratch_types=[
            pltpu.VMEM((NUM_FEATURES, CHUNK), jnp.int32),
            pltpu.VMEM((NUM_FEATURES, CHUNK), jnp.int32),
            pltpu.VMEM((2, CHUNK, EMB_DIM), jnp.float32),
            pltpu.SemaphoreType.DMA,
        ],
        compiler_params=pltpu.CompilerParams(use_tc_tiling_on_sc=False),
    )
    return k(idx32, lin_tables)


def kernel(indices, tables):
    idx32 = indices.astype(jnp.int32)
    tabT = tables.transpose(0, 2, 1)
    lin = _relayout(tabT)
    return _gather(idx32, lin.reshape(NUM_FEATURES * VOCAB, EMB_DIM))


# speedup vs baseline: 1.0618x; 1.0618x over previous
import jax
import jax.numpy as jnp
from jax import lax
from jax.experimental import pallas as pl
from jax.experimental.pallas import tpu as pltpu
from jax.experimental.pallas import tpu_sc as plsc

NUM_FEATURES = 26
BATCH = 4096
VOCAB = 100000
EMB_DIM = 32

NC = 2
NS = 16
LANES = 16
NW = NC * NS
CHUNK = BATCH // NW

NTILES = 782
NTILES_USED = 782
TOTAL_TC = NUM_FEATURES * NTILES
PER_W = [TOTAL_TC // NW + (1 if w < TOTAL_TC % NW else 0) for w in range(NW)]
LO_W = [0] * NW
for w in range(1, NW):
    LO_W[w] = LO_W[w - 1] + PER_W[w - 1]


def _relayout_body(tab_hbm, lin_hbm, stage_v, col_v, sem, osem):
    c = lax.axis_index("c")
    s = lax.axis_index("s")
    wid = s * NC + c

    lo = jnp.int32(0)
    hi = jnp.int32(0)
    for w in range(NW):
        lo = jnp.where(wid == w, jnp.int32(LO_W[w]), lo)
        hi = jnp.where(wid == w, jnp.int32(LO_W[w] + PER_W[w]), hi)

    iota = lax.iota(jnp.int32, LANES)

    def fetch(tc, buf):
        f = tc // NTILES
        t = tc % NTILES
        toff = pl.multiple_of(t * 128, 128)
        for g in range(4):
            pltpu.async_copy(
                tab_hbm.at[f, pl.ds(8 * g, 8), pl.ds(toff, 128)],
                stage_v.at[buf, pl.ds(8 * g, 8)],
                sem,
            )

    def wait_fetch(tc, buf):
        f = tc // NTILES
        t = tc % NTILES
        toff = pl.multiple_of(t * 128, 128)
        for g in range(4):
            pltpu.make_async_copy(
                tab_hbm.at[f, pl.ds(8 * g, 8), pl.ds(toff, 128)],
                stage_v.at[buf, pl.ds(8 * g, 8)],
                sem,
            ).wait()

    fetch(lo, 0)

    def body(i, _):
        tc = lo + i
        buf = i % 2

        @pl.when(tc + 1 < hi)
        def _():
            fetch(tc + 1, 1 - buf)

        wait_fetch(tc, buf)

        @pl.when(i >= 2)
        def _():
            tcp = tc - 2
            fp = tcp // NTILES
            tp = tcp % NTILES
            rp = pl.multiple_of(fp * (VOCAB // 4) + 32 * tp, 8)

            @pl.when(tp < NTILES - 1)
            def _():
                pltpu.make_async_copy(
                    col_v.at[buf], lin_hbm.at[pl.ds(rp, 32), :], osem
                ).wait()

            @pl.when(tp == NTILES - 1)
            def _():
                pltpu.make_async_copy(
                    col_v.at[buf, pl.ds(0, 8), :], lin_hbm.at[pl.ds(rp, 8), :], osem
                ).wait()

        for j in range(32):
            for k in range(8):
                g = plsc.load_gather(
                    stage_v,
                    [
                        jnp.full((LANES,), buf, dtype=jnp.int32),
                        iota + 16 * (k % 2),
                        jnp.full((LANES,), 4 * j + k // 2, dtype=jnp.int32),
                    ],
                )
                col_v[buf, j, pl.ds(16 * k, 16)] = g

        f = tc // NTILES
        t = tc % NTILES
        r0 = pl.multiple_of(f * (VOCAB // 4) + 32 * t, 8)

        @pl.when(t < NTILES - 1)
        def _():
            pltpu.async_copy(col_v.at[buf], lin_hbm.at[pl.ds(r0, 32), :], osem)

        @pl.when(t == NTILES - 1)
        def _():
            pltpu.async_copy(
                col_v.at[buf, pl.ds(0, 8), :], lin_hbm.at[pl.ds(r0, 8), :], osem
            )

        return 0

    n = hi - lo
    lax.fori_loop(0, n, body, 0, unroll=False)

    def drain_one(i, _):
        tc = lo + i
        f = tc // NTILES
        t = tc % NTILES
        r0 = pl.multiple_of(f * (VOCAB // 4) + 32 * t, 8)
        buf = i % 2

        @pl.when(t < NTILES - 1)
        def _():
            pltpu.make_async_copy(
                col_v.at[buf], lin_hbm.at[pl.ds(r0, 32), :], osem
            ).wait()

        @pl.when(t == NTILES - 1)
        def _():
            pltpu.make_async_copy(
                col_v.at[buf, pl.ds(0, 8), :], lin_hbm.at[pl.ds(r0, 8), :], osem
            ).wait()

        return 0

    lax.fori_loop(n - 2, n, drain_one, 0, unroll=False)


@jax.jit
def _relayout(tabT):
    mesh = plsc.VectorSubcoreMesh(
        core_axis_name="c", subcore_axis_name="s", num_cores=NC, num_subcores=NS
    )
    k = pl.kernel(
        _relayout_body,
        out_type=jax.ShapeDtypeStruct((VOCAB * NUM_FEATURES // 4, 128), jnp.float32),
        mesh=mesh,
        scratch_types=[
            pltpu.VMEM((2, EMB_DIM, 128), jnp.float32),
            pltpu.VMEM((2, EMB_DIM, 128), jnp.float32),
            pltpu.SemaphoreType.DMA,
            pltpu.SemaphoreType.DMA,
        ],
        compiler_params=pltpu.CompilerParams(
            use_tc_tiling_on_sc=True, needs_layout_passes=False
        ),
    )
    return k(tabT)


def _gather_body(idx_hbm, tab_hbm, out_hbm, idx_v, gidx_v, rows_v, sem):
    c = lax.axis_index("c")
    s = lax.axis_index("s")
    wid = s * NC + c
    base_b = wid * CHUNK

    pltpu.sync_copy(idx_hbm.at[:, pl.ds(base_b, CHUNK)], idx_v)

    for f in range(NUM_FEATURES):
        off = jnp.full((LANES,), f * VOCAB, dtype=jnp.int32)
        for k in range(CHUNK // LANES):
            sl = pl.ds(k * LANES, LANES)
            gidx_v[f, sl] = idx_v[f, sl] + off

    copies = [None, None]
    for f in range(NUM_FEATURES):
        b = f % 2
        if copies[b] is not None:
            copies[b].wait()
            pltpu.sync_copy(
                rows_v.at[b],
                out_hbm.at[pl.ds((f - 2) * BATCH + base_b, CHUNK)],
            )
        copies[b] = pltpu.async_copy(tab_hbm.at[gidx_v.at[f]], rows_v.at[b], sem)
    for f in (NUM_FEATURES - 2, NUM_FEATURES - 1):
        b = f % 2
        copies[b].wait()
        pltpu.sync_copy(
            rows_v.at[b],
            out_hbm.at[pl.ds(f * BATCH + base_b, CHUNK)],
        )


@jax.jit
def _gather(idx32, lin_tables):
    mesh = plsc.VectorSubcoreMesh(
        core_axis_name="c", subcore_axis_name="s", num_cores=NC, num_subcores=NS
    )
    k = pl.kernel(
        _gather_body,
        out_type=jax.ShapeDtypeStruct((NUM_FEATURES * BATCH, EMB_DIM), jnp.float32),
        mesh=mesh,
        scratch_types=[
            pltpu.VMEM((NUM_FEATURES, CHUNK), jnp.int32),
            pltpu.VMEM((NUM_FEATURES, CHUNK), jnp.int32),
            pltpu.VMEM((2, CHUNK, EMB_DIM), jnp.float32),
            pltpu.SemaphoreType.DMA,
        ],
        compiler_params=pltpu.CompilerParams(use_tc_tiling_on_sc=False),
    )
    return k(idx32, lin_tables)


def kernel(indices, tables):
    idx32 = indices.astype(jnp.int32)
    tabT = tables.transpose(0, 2, 1)
    lin = _relayout(tabT)
    return _gather(idx32, lin.reshape(NUM_FEATURES * VOCAB, EMB_DIM))


# speedup vs baseline: 1.7609x; 1.6584x over previous
import functools

import jax
import jax.numpy as jnp
from jax import lax
from jax.experimental import pallas as pl
from jax.experimental.pallas import tpu as pltpu
from jax.experimental.pallas import tpu_sc as plsc

NUM_FEATURES = 26
BATCH = 4096
VOCAB = 100000
EMB_DIM = 32

NC = 2
NS = 16
LANES = 16
NW = NC * NS
CHUNK = BATCH // NW


def _emb_body(idx_hbm, tab_hbm, out_hbm, idx_v, gidx_v, rows_v, sem):
    c = lax.axis_index("c")
    s = lax.axis_index("s")
    wid = s * NC + c
    base_b = wid * CHUNK

    pltpu.sync_copy(idx_hbm.at[:, pl.ds(base_b, CHUNK)], idx_v)

    for f in range(NUM_FEATURES):
        off = jnp.full((LANES,), f * VOCAB, dtype=jnp.int32)
        for k in range(CHUNK // LANES):
            sl = pl.ds(k * LANES, LANES)
            gidx_v[f, sl] = idx_v[f, sl] + off

    copies = [None, None]
    for f in range(NUM_FEATURES):
        b = f % 2
        if copies[b] is not None:
            copies[b].wait()
            pltpu.sync_copy(
                rows_v.at[b],
                out_hbm.at[pl.ds((f - 2) * BATCH + base_b, CHUNK)],
            )
        copies[b] = pltpu.async_copy(tab_hbm.at[gidx_v.at[f]], rows_v.at[b], sem)
    for f in (NUM_FEATURES - 2, NUM_FEATURES - 1):
        b = f % 2
        copies[b].wait()
        pltpu.sync_copy(
            rows_v.at[b],
            out_hbm.at[pl.ds(f * BATCH + base_b, CHUNK)],
        )


@jax.jit
def _run(idx32, flat_tables):
    mesh = plsc.VectorSubcoreMesh(
        core_axis_name="c", subcore_axis_name="s", num_cores=NC, num_subcores=NS
    )
    k = pl.kernel(
        _emb_body,
        out_type=jax.ShapeDtypeStruct((NUM_FEATURES * BATCH, EMB_DIM), jnp.float32),
        mesh=mesh,
        scratch_types=[
            pltpu.VMEM((NUM_FEATURES, CHUNK), jnp.int32),
            pltpu.VMEM((NUM_FEATURES, CHUNK), jnp.int32),
            pltpu.VMEM((2, CHUNK, EMB_DIM), jnp.float32),
            pltpu.SemaphoreType.DMA,
        ],
        compiler_params=pltpu.CompilerParams(use_tc_tiling_on_sc=False),
    )
    return k(idx32, flat_tables)


def kernel(indices, tables):
    idx32 = indices.astype(jnp.int32)
    flat_tables = tables.reshape(NUM_FEATURES * VOCAB, EMB_DIM)
    return _run(idx32, flat_tables)
